# baseline (speedup 1.0000x reference)
import jax
import jax.numpy as jnp
from jax import lax
from jax.experimental import pallas as pl
from jax.experimental.pallas import tpu as pltpu
from jax.experimental.pallas import tpu_sc as plsc

BATCH = 16384
HIST = 50
D = 32
V = 1_000_000
VR = V // 4

_info = plsc.get_sparse_core_info()
NC = _info.num_cores
NS = _info.num_subcores
NW = NC * NS

CH1 = 512
N1 = V // CH1
N1_FULL = 1953
TAIL_C0 = N1_FULL * CH1
P1_ELEMS = D * CH1
P1_GROUPS = P1_ELEMS // 16

C2 = 128
TASKS = HIST * (BATCH // C2)
TASKS_PER_W = TASKS // NW
BC_PER_H = BATCH // C2


def _body(idx_hbm, tab_hbm, aux_hbm, out_hbm, tabrm_hbm,
          a_v, b_v, dall, call,
          i_v0, i_v1, j_v0, j_v1, cq_v0, cq_v1, g_v0, g_v1, o_v0, o_v1,
          sem0, sem1, sem_x):
    cid = lax.axis_index("c")
    sid = lax.axis_index("s")
    wid = sid * NC + cid

    i16 = lax.iota(jnp.int32, 16)

    def fill(g, carry):
        p = i16 + g * 16
        dall[pl.ds(g * 16, 16)] = lax.rem(p, jnp.int32(D))
        call[pl.ds(g * 16, 16)] = lax.div(p, jnp.int32(D))
        return carry

    lax.fori_loop(0, P1_GROUPS, fill, 0)

    def transpose_chunk(n_groups, n_rows, row0):
        def tg(g, carry):
            dvec = dall[pl.ds(g * 16, 16)]
            cvec = call[pl.ds(g * 16, 16)]
            vals = plsc.load_gather(a_v, [dvec, cvec])
            b_v[lax.div(g, 8), pl.ds(lax.rem(g, 8) * 16, 16)] = vals
            return carry

        lax.fori_loop(0, n_groups, tg, 0)
        pltpu.sync_copy(b_v.at[pl.ds(0, n_rows)], tabrm_hbm.at[pl.ds(row0, n_rows)])

    n_mine = 61 + jnp.where(wid + 61 * NW < N1_FULL, 1, 0)

    def p1_step(j, carry):
        k = wid + j * NW
        c0 = k * CH1
        pltpu.sync_copy(tab_hbm.at[:, pl.ds(c0, CH1)], a_v)
        transpose_chunk(P1_GROUPS, 128, k * 128)
        return carry

    lax.fori_loop(0, n_mine, p1_step, 0)

    @pl.when(wid == 1)
    def _tail():
        pltpu.sync_copy(aux_hbm, b_v.at[pl.ds(0, 16)])
        pltpu.sync_copy(b_v.at[pl.ds(0, 16)], tabrm_hbm.at[pl.ds(VR - 16, 16)])

    plsc.subcore_barrier()

    @pl.when(sid == 0)
    def _handshake():
        pl.semaphore_signal(sem_x, 1, core_index=1 - cid)
        pl.semaphore_wait(sem_x, 1)

    plsc.subcore_barrier()

    def start_task(m, i_v, j_v, cq_v, g_v, sem):
        t = wid * TASKS_PER_W + m
        h = lax.div(t, BC_PER_H)
        b0 = lax.rem(t, BC_PER_H) * C2
        pltpu.sync_copy(idx_hbm.at[h, pl.ds(b0, C2)], i_v)

        def prep(g, carry):
            iv = i_v[pl.ds(g * 16, 16)]
            j_v[pl.ds(g * 16, 16)] = lax.shift_right_logical(iv, 2)
            q = lax.bitwise_and(iv, jnp.int32(3))
            cq_v[pl.ds(g * 16, 16)] = q * 32
            return carry

        lax.fori_loop(0, C2 // 16, prep, 0)
        return pltpu.async_copy(tabrm_hbm.at[j_v], g_v, sem)

    def finish_task(m, cq_v, g_v, o_v, copy):
        copy.wait()
        for gi in range(C2 // 16):
            rvec = i16 + gi * 16
            cq = cq_v[pl.ds(gi * 16, 16)]
            for d in range(D):
                vals = plsc.load_gather(g_v, [rvec, cq + d])
                o_v[d, pl.ds(gi * 16, 16)] = vals
        t = wid * TASKS_PER_W + m
        h = lax.div(t, BC_PER_H)
        b0 = lax.rem(t, BC_PER_H) * C2
        pltpu.sync_copy(o_v, out_hbm.at[h, :, pl.ds(b0, C2)])

    c0 = start_task(0, i_v0, j_v0, cq_v0, g_v0, sem0)

    def p2_step(m2, carry):
        m = m2 * 2
        c1 = start_task(m + 1, i_v1, j_v1, cq_v1, g_v1, sem1)
        finish_task(m, cq_v0, g_v0, o_v0, pltpu.make_async_copy(tabrm_hbm.at[j_v0], g_v0, sem0))

        @pl.when(m + 2 < TASKS_PER_W)
        def _():
            start_task(m + 2, i_v0, j_v0, cq_v0, g_v0, sem0)

        finish_task(m + 1, cq_v1, g_v1, o_v1, pltpu.make_async_copy(tabrm_hbm.at[j_v1], g_v1, sem1))
        return carry

    lax.fori_loop(0, TASKS_PER_W // 2, p2_step, 0)
    del c0


@jax.jit
def kernel(input, params):
    idx_t = input.T
    tab_t = params.T
    aux = jnp.reshape(params[TAIL_C0:], (16, 128))
    mesh = plsc.VectorSubcoreMesh(core_axis_name="c", subcore_axis_name="s")
    out_p, _ = pl.kernel(
        _body,
        mesh=mesh,
        out_type=[
            jax.ShapeDtypeStruct((HIST, D, BATCH), jnp.float32),
            jax.ShapeDtypeStruct((VR, 128), jnp.float32),
        ],
        scratch_types=[
            pltpu.VMEM((D, CH1), jnp.float32),
            pltpu.VMEM((128, 128), jnp.float32),
            pltpu.VMEM((P1_ELEMS,), jnp.int32),
            pltpu.VMEM((P1_ELEMS,), jnp.int32),
            pltpu.VMEM((C2,), jnp.int32),
            pltpu.VMEM((C2,), jnp.int32),
            pltpu.VMEM((C2,), jnp.int32),
            pltpu.VMEM((C2,), jnp.int32),
            pltpu.VMEM((C2,), jnp.int32),
            pltpu.VMEM((C2,), jnp.int32),
            pltpu.VMEM((C2, 128), jnp.float32),
            pltpu.VMEM((C2, 128), jnp.float32),
            pltpu.VMEM((D, C2), jnp.float32),
            pltpu.VMEM((D, C2), jnp.float32),
            pltpu.SemaphoreType.DMA,
            pltpu.SemaphoreType.DMA,
            pltpu.SemaphoreType.REGULAR,
        ],
        compiler_params=pltpu.CompilerParams(
            use_tc_tiling_on_sc=True, needs_layout_passes=False
        ),
    )(idx_t, tab_t, aux)
    return jnp.transpose(out_p, (2, 0, 1))


# speedup vs baseline: 1.5878x; 1.5878x over previous
import jax
import jax.numpy as jnp
from jax import lax
from jax.experimental import pallas as pl
from jax.experimental.pallas import tpu as pltpu
from jax.experimental.pallas import tpu_sc as plsc

BATCH = 16384
HIST = 50
D = 32
V = 1_000_000
VR = V // 4

_info = plsc.get_sparse_core_info()
NC = _info.num_cores
NS = _info.num_subcores
NW = NC * NS

CH1 = 256
N1_FULL = 3906
TAIL_C0 = N1_FULL * CH1
P1_PAIRS = 61
P1_GROUPS = D * CH1 // 16 // 16

C2 = 256
BC_PER_H = BATCH // C2
TASKS_PER_W = HIST * BC_PER_H // NW


def _body(idx_hbm, tab_hbm, aux_hbm, out_hbm, tabrm_hbm,
          a_v0, a_v1, b_v0, b_v1,
          i_v0, i_v1, j_v0, j_v1, cq_v0, cq_v1, g_v0, g_v1, o_v0, o_v1,
          semA0, semA1, semB0, semB1, semI0, semI1, sem_x):
    cid = lax.axis_index("c")
    sid = lax.axis_index("s")
    wid = sid * NC + cid

    i16 = lax.iota(jnp.int32, 16)

    def p1_in(j, a_v, sem):
        c0 = (wid + j * NW) * CH1
        return pltpu.async_copy(tab_hbm.at[:, pl.ds(c0, CH1)], a_v, sem)

    def p1_transpose(a_v, b_v):
        def tg(g, carry):
            for gg in range(16):
                dvec = i16 + 16 * (gg % 2)
                cvec = jnp.full((16,), 8 * g + gg // 2, jnp.int32)
                vals = plsc.load_gather(a_v, [dvec, cvec])
                b_v[2 * g + gg // 8, pl.ds(16 * (gg % 8), 16)] = vals
            return carry

        lax.fori_loop(0, P1_GROUPS, tg, 0)

    def p1_finish(j, k, a_v, b_v, semA, semB):
        c0 = (wid + j * NW) * CH1
        pltpu.make_async_copy(tab_hbm.at[:, pl.ds(c0, CH1)], a_v, semA).wait()

        @pl.when(k >= 1)
        def _():
            pltpu.make_async_copy(b_v, tabrm_hbm.at[pl.ds(0, 64)], semB).wait()

        p1_transpose(a_v, b_v)
        return pltpu.async_copy(b_v, tabrm_hbm.at[pl.ds((wid + j * NW) * 64, 64)], semB)

    p1_in(0, a_v0, semA0)

    def p1_step(k, carry):
        p1_in(2 * k + 1, a_v1, semA1)
        p1_finish(2 * k, k, a_v0, b_v0, semA0, semB0)

        @pl.when(2 * k + 2 < 2 * P1_PAIRS)
        def _():
            p1_in(2 * k + 2, a_v0, semA0)

        p1_finish(2 * k + 1, k, a_v1, b_v1, semA1, semB1)
        return carry

    lax.fori_loop(0, P1_PAIRS, p1_step, 0)
    pltpu.make_async_copy(b_v0, tabrm_hbm.at[pl.ds(0, 64)], semB0).wait()
    pltpu.make_async_copy(b_v1, tabrm_hbm.at[pl.ds(0, 64)], semB1).wait()

    @pl.when(wid < 2)
    def _extra():
        j = 2 * P1_PAIRS
        pltpu.async_copy(tab_hbm.at[:, pl.ds((wid + j * NW) * CH1, CH1)], a_v0, semA0).wait()
        p1_transpose(a_v0, b_v0)
        pltpu.async_copy(b_v0, tabrm_hbm.at[pl.ds((wid + j * NW) * 64, 64)], semB0).wait()

    @pl.when(wid == 2)
    def _tail():
        pltpu.async_copy(aux_hbm, b_v0.at[pl.ds(0, 16)], semA0).wait()
        pltpu.async_copy(b_v0.at[pl.ds(0, 16)], tabrm_hbm.at[pl.ds(VR - 16, 16)], semB0).wait()

    plsc.subcore_barrier()

    @pl.when(sid == 0)
    def _handshake():
        pl.semaphore_signal(sem_x, 1, core_index=1 - cid)
        pl.semaphore_wait(sem_x, 1)

    plsc.subcore_barrier()

    def task_slices(m):
        t = wid * TASKS_PER_W + m
        h = lax.div(t, BC_PER_H)
        b0 = lax.rem(t, BC_PER_H) * C2
        return h, b0

    def p2_idx(m, i_v, semI):
        h, b0 = task_slices(m)
        return pltpu.async_copy(idx_hbm.at[h, pl.ds(b0, C2)], i_v, semI)

    def p2_prep_gather(i_v, j_v, cq_v, g_v, semA):
        for g in range(C2 // 16):
            iv = i_v[pl.ds(g * 16, 16)]
            j_v[pl.ds(g * 16, 16)] = lax.shift_right_logical(iv, 2)
            cq_v[pl.ds(g * 16, 16)] = lax.bitwise_and(iv, jnp.int32(3)) * 32
        return pltpu.async_copy(tabrm_hbm.at[j_v], g_v, semA)

    def p2_extract(m, k, j_v, cq_v, g_v, o_v, semA, semB):
        pltpu.make_async_copy(tabrm_hbm.at[j_v], g_v, semA).wait()

        @pl.when(k >= 1)
        def _():
            pltpu.make_async_copy(o_v, out_hbm.at[0, :, pl.ds(0, C2)], semB).wait()

        def eg(gi, carry):
            rvec = i16 + gi * 16
            cq = cq_v[pl.ds(gi * 16, 16)]
            for d in range(D):
                o_v[d, pl.ds(gi * 16, 16)] = plsc.load_gather(g_v, [rvec, cq + d])
            return carry

        lax.fori_loop(0, C2 // 16, eg, 0)
        h, b0 = task_slices(m)
        return pltpu.async_copy(o_v, out_hbm.at[h, :, pl.ds(b0, C2)], semB)

    p2_idx(0, i_v0, semI0).wait()
    p2_prep_gather(i_v0, j_v0, cq_v0, g_v0, semA0)
    p2_idx(1, i_v1, semI1).wait()
    p2_prep_gather(i_v1, j_v1, cq_v1, g_v1, semA1)

    def p2_step(k, carry):
        m = 2 * k
        cont = m + 2 < TASKS_PER_W

        @pl.when(cont)
        def _():
            p2_idx(m + 2, i_v0, semI0)

        p2_extract(m, k, j_v0, cq_v0, g_v0, o_v0, semA0, semB0)

        @pl.when(cont)
        def _():
            pltpu.make_async_copy(idx_hbm.at[0, pl.ds(0, C2)], i_v0, semI0).wait()
            p2_prep_gather(i_v0, j_v0, cq_v0, g_v0, semA0)

        @pl.when(cont)
        def _():
            p2_idx(m + 3, i_v1, semI1)

        p2_extract(m + 1, k, j_v1, cq_v1, g_v1, o_v1, semA1, semB1)

        @pl.when(cont)
        def _():
            pltpu.make_async_copy(idx_hbm.at[0, pl.ds(0, C2)], i_v1, semI1).wait()
            p2_prep_gather(i_v1, j_v1, cq_v1, g_v1, semA1)

        return carry

    lax.fori_loop(0, TASKS_PER_W // 2, p2_step, 0)
    pltpu.make_async_copy(o_v0, out_hbm.at[0, :, pl.ds(0, C2)], semB0).wait()
    pltpu.make_async_copy(o_v1, out_hbm.at[0, :, pl.ds(0, C2)], semB1).wait()


@jax.jit
def kernel(input, params):
    idx_t = input.T
    tab_t = params.T
    aux = jnp.reshape(params[TAIL_C0:], (16, 128))
    mesh = plsc.VectorSubcoreMesh(core_axis_name="c", subcore_axis_name="s")
    out_p, _ = pl.kernel(
        _body,
        mesh=mesh,
        out_type=[
            jax.ShapeDtypeStruct((HIST, D, BATCH), jnp.float32),
            jax.ShapeDtypeStruct((VR, 128), jnp.float32),
        ],
        scratch_types=[
            pltpu.VMEM((D, CH1), jnp.float32),
            pltpu.VMEM((D, CH1), jnp.float32),
            pltpu.VMEM((64, 128), jnp.float32),
            pltpu.VMEM((64, 128), jnp.float32),
            pltpu.VMEM((C2,), jnp.int32),
            pltpu.VMEM((C2,), jnp.int32),
            pltpu.VMEM((C2,), jnp.int32),
            pltpu.VMEM((C2,), jnp.int32),
            pltpu.VMEM((C2,), jnp.int32),
            pltpu.VMEM((C2,), jnp.int32),
            pltpu.VMEM((C2, 128), jnp.float32),
            pltpu.VMEM((C2, 128), jnp.float32),
            pltpu.VMEM((D, C2), jnp.float32),
            pltpu.VMEM((D, C2), jnp.float32),
            pltpu.SemaphoreType.DMA,
            pltpu.SemaphoreType.DMA,
            pltpu.SemaphoreType.DMA,
            pltpu.SemaphoreType.DMA,
            pltpu.SemaphoreType.DMA,
            pltpu.SemaphoreType.DMA,
            pltpu.SemaphoreType.REGULAR,
        ],
        compiler_params=pltpu.CompilerParams(
            use_tc_tiling_on_sc=True, needs_layout_passes=False
        ),
    )(idx_t, tab_t, aux)
    return jnp.transpose(out_p, (2, 0, 1))


# speedup vs baseline: 1.9682x; 1.2396x over previous
import jax
import jax.numpy as jnp
from jax import lax
from jax.experimental import pallas as pl
from jax.experimental.pallas import tpu as pltpu
from jax.experimental.pallas import tpu_sc as plsc

BATCH = 16384
HIST = 50
D = 32
V = 1_000_000
VR = V // 4

_info = plsc.get_sparse_core_info()
NC = _info.num_cores
NS = _info.num_subcores
NW = NC * NS

C2 = 256
BC_PER_H = BATCH // C2
TASKS_PER_W = HIST * BC_PER_H // NW


def _body(idx_hbm, tab_hbm, out_hbm,
          i_v0, i_v1, j_v0, j_v1, cq_v0, cq_v1, g_v0, g_v1, o_v0, o_v1,
          semA0, semA1, semB0, semB1, semI0, semI1):
    cid = lax.axis_index("c")
    sid = lax.axis_index("s")
    wid = sid * NC + cid

    i16 = lax.iota(jnp.int32, 16)

    def task_slices(m):
        t = wid * TASKS_PER_W + m
        h = lax.div(t, BC_PER_H)
        b0 = lax.rem(t, BC_PER_H) * C2
        return h, b0

    def p2_idx(m, i_v, semI):
        h, b0 = task_slices(m)
        return pltpu.async_copy(idx_hbm.at[h, pl.ds(b0, C2)], i_v, semI)

    def p2_prep_gather(i_v, j_v, cq_v, g_v, semA):
        for g in range(C2 // 16):
            iv = i_v[pl.ds(g * 16, 16)]
            j_v[pl.ds(g * 16, 16)] = lax.shift_right_logical(iv, 2)
            cq_v[pl.ds(g * 16, 16)] = lax.bitwise_and(iv, jnp.int32(3)) * 32
        return pltpu.async_copy(tab_hbm.at[j_v], g_v, semA)

    def p2_extract(m, k, j_v, cq_v, g_v, o_v, semA, semB):
        pltpu.make_async_copy(tab_hbm.at[j_v], g_v, semA).wait()

        @pl.when(k >= 1)
        def _():
            pltpu.make_async_copy(o_v, out_hbm.at[0, :, pl.ds(0, C2)], semB).wait()

        def eg(gi, carry):
            rvec = i16 + gi * 16
            cq = cq_v[pl.ds(gi * 16, 16)]
            for d in range(D):
                o_v[d, pl.ds(gi * 16, 16)] = plsc.load_gather(g_v, [rvec, cq + d])
            return carry

        lax.fori_loop(0, C2 // 16, eg, 0)
        h, b0 = task_slices(m)
        return pltpu.async_copy(o_v, out_hbm.at[h, :, pl.ds(b0, C2)], semB)

    p2_idx(0, i_v0, semI0).wait()
    p2_prep_gather(i_v0, j_v0, cq_v0, g_v0, semA0)
    p2_idx(1, i_v1, semI1).wait()
    p2_prep_gather(i_v1, j_v1, cq_v1, g_v1, semA1)

    def p2_step(k, carry):
        m = 2 * k
        cont = m + 2 < TASKS_PER_W

        @pl.when(cont)
        def _():
            p2_idx(m + 2, i_v0, semI0)

        p2_extract(m, k, j_v0, cq_v0, g_v0, o_v0, semA0, semB0)

        @pl.when(cont)
        def _():
            pltpu.make_async_copy(idx_hbm.at[0, pl.ds(0, C2)], i_v0, semI0).wait()
            p2_prep_gather(i_v0, j_v0, cq_v0, g_v0, semA0)

        @pl.when(cont)
        def _():
            p2_idx(m + 3, i_v1, semI1)

        p2_extract(m + 1, k, j_v1, cq_v1, g_v1, o_v1, semA1, semB1)

        @pl.when(cont)
        def _():
            pltpu.make_async_copy(idx_hbm.at[0, pl.ds(0, C2)], i_v1, semI1).wait()
            p2_prep_gather(i_v1, j_v1, cq_v1, g_v1, semA1)

        return carry

    lax.fori_loop(0, TASKS_PER_W // 2, p2_step, 0)
    pltpu.make_async_copy(o_v0, out_hbm.at[0, :, pl.ds(0, C2)], semB0).wait()
    pltpu.make_async_copy(o_v1, out_hbm.at[0, :, pl.ds(0, C2)], semB1).wait()


@jax.jit
def kernel(input, params):
    idx_t = input.T
    tab4 = jnp.reshape(params, (VR, 128))
    mesh = plsc.VectorSubcoreMesh(core_axis_name="c", subcore_axis_name="s")
    out_p = pl.kernel(
        _body,
        mesh=mesh,
        out_type=jax.ShapeDtypeStruct((HIST, D, BATCH), jnp.float32),
        scratch_types=[
            pltpu.VMEM((C2,), jnp.int32),
            pltpu.VMEM((C2,), jnp.int32),
            pltpu.VMEM((C2,), jnp.int32),
            pltpu.VMEM((C2,), jnp.int32),
            pltpu.VMEM((C2,), jnp.int32),
            pltpu.VMEM((C2,), jnp.int32),
            pltpu.VMEM((C2, 128), jnp.float32),
            pltpu.VMEM((C2, 128), jnp.float32),
            pltpu.VMEM((D, C2), jnp.float32),
            pltpu.VMEM((D, C2), jnp.float32),
            pltpu.SemaphoreType.DMA,
            pltpu.SemaphoreType.DMA,
            pltpu.SemaphoreType.DMA,
            pltpu.SemaphoreType.DMA,
            pltpu.SemaphoreType.DMA,
            pltpu.SemaphoreType.DMA,
        ],
        compiler_params=pltpu.CompilerParams(
            use_tc_tiling_on_sc=True, needs_layout_passes=False
        ),
    )(idx_t, tab4)
    return jnp.transpose(out_p, (2, 0, 1))


# speedup vs baseline: 3.0111x; 1.5299x over previous
import jax
import jax.numpy as jnp
from jax import lax
from jax.experimental import pallas as pl
from jax.experimental.pallas import tpu as pltpu
from jax.experimental.pallas import tpu_sc as plsc

BATCH = 16384
HIST = 50
D = 32
V = 1_000_000
VR = V // 4

_info = plsc.get_sparse_core_info()
NC = _info.num_cores
NS = _info.num_subcores
NW = NC * NS

C2 = 256
BC_PER_H = BATCH // C2
TASKS_PER_W = HIST * BC_PER_H // NW


def _body(idx_hbm, tab_hbm, out_hbm,
          i_v0, i_v1, j_v0, j_v1, cq_v0, cq_v1, g_v0, g_v1, o_v0, o_v1,
          semA0, semA1, semB0, semB1, semI0, semI1):
    cid = lax.axis_index("c")
    sid = lax.axis_index("s")
    wid = sid * NC + cid

    i16 = lax.iota(jnp.int32, 16)

    def task_slices(m):
        t = wid * TASKS_PER_W + m
        h = lax.div(t, BC_PER_H)
        b0 = lax.rem(t, BC_PER_H) * C2
        return h, b0

    def p2_idx(m, i_v, semI):
        h, b0 = task_slices(m)
        return pltpu.async_copy(idx_hbm.at[h, pl.ds(b0, C2)], i_v, semI)

    def p2_prep_gather(i_v, j_v, cq_v, g_v, semA):
        for g in range(C2 // 16):
            iv = i_v[pl.ds(g * 16, 16)]
            j_v[pl.ds(g * 16, 16)] = lax.shift_right_logical(iv, 2)
            cq_v[pl.ds(g * 16, 16)] = lax.bitwise_and(iv, jnp.int32(3)) * 32
        return pltpu.async_copy(tab_hbm.at[j_v], g_v, semA)

    def p2_extract(m, k, j_v, cq_v, g_v, o_v, semA, semB):
        pltpu.make_async_copy(tab_hbm.at[j_v], g_v, semA).wait()

        @pl.when(k >= 1)
        def _():
            pltpu.make_async_copy(o_v, out_hbm.at[0, :, pl.ds(0, C2)], semB).wait()

        def eg(gi, carry):
            rvec = i16 + gi * 16
            cq = cq_v[pl.ds(gi * 16, 16)]
            for dd in range(D):
                dvec = lax.bitwise_and(i16 + dd, jnp.int32(D - 1))
                vals = plsc.load_gather(g_v, [rvec, cq + dvec])
                plsc.store_scatter(o_v, [dvec, rvec], vals)
            return carry

        lax.fori_loop(0, C2 // 16, eg, 0)
        h, b0 = task_slices(m)
        return pltpu.async_copy(o_v, out_hbm.at[h, :, pl.ds(b0, C2)], semB)

    p2_idx(0, i_v0, semI0).wait()
    p2_prep_gather(i_v0, j_v0, cq_v0, g_v0, semA0)
    p2_idx(1, i_v1, semI1).wait()
    p2_prep_gather(i_v1, j_v1, cq_v1, g_v1, semA1)

    def p2_step(k, carry):
        m = 2 * k
        cont = m + 2 < TASKS_PER_W

        @pl.when(cont)
        def _():
            p2_idx(m + 2, i_v0, semI0)

        p2_extract(m, k, j_v0, cq_v0, g_v0, o_v0, semA0, semB0)

        @pl.when(cont)
        def _():
            pltpu.make_async_copy(idx_hbm.at[0, pl.ds(0, C2)], i_v0, semI0).wait()
            p2_prep_gather(i_v0, j_v0, cq_v0, g_v0, semA0)

        @pl.when(cont)
        def _():
            p2_idx(m + 3, i_v1, semI1)

        p2_extract(m + 1, k, j_v1, cq_v1, g_v1, o_v1, semA1, semB1)

        @pl.when(cont)
        def _():
            pltpu.make_async_copy(idx_hbm.at[0, pl.ds(0, C2)], i_v1, semI1).wait()
            p2_prep_gather(i_v1, j_v1, cq_v1, g_v1, semA1)

        return carry

    lax.fori_loop(0, TASKS_PER_W // 2, p2_step, 0)
    pltpu.make_async_copy(o_v0, out_hbm.at[0, :, pl.ds(0, C2)], semB0).wait()
    pltpu.make_async_copy(o_v1, out_hbm.at[0, :, pl.ds(0, C2)], semB1).wait()


@jax.jit
def kernel(input, params):
    idx_t = input.T
    tab4 = jnp.reshape(params, (VR, 128))
    mesh = plsc.VectorSubcoreMesh(core_axis_name="c", subcore_axis_name="s")
    out_p = pl.kernel(
        _body,
        mesh=mesh,
        out_type=jax.ShapeDtypeStruct((HIST, D, BATCH), jnp.float32),
        scratch_types=[
            pltpu.VMEM((C2,), jnp.int32),
            pltpu.VMEM((C2,), jnp.int32),
            pltpu.VMEM((C2,), jnp.int32),
            pltpu.VMEM((C2,), jnp.int32),
            pltpu.VMEM((C2,), jnp.int32),
            pltpu.VMEM((C2,), jnp.int32),
            pltpu.VMEM((C2, 128), jnp.float32),
            pltpu.VMEM((C2, 128), jnp.float32),
            pltpu.VMEM((D, C2), jnp.float32),
            pltpu.VMEM((D, C2), jnp.float32),
            pltpu.SemaphoreType.DMA,
            pltpu.SemaphoreType.DMA,
            pltpu.SemaphoreType.DMA,
            pltpu.SemaphoreType.DMA,
            pltpu.SemaphoreType.DMA,
            pltpu.SemaphoreType.DMA,
        ],
        compiler_params=pltpu.CompilerParams(
            use_tc_tiling_on_sc=True, needs_layout_passes=False
        ),
    )(idx_t, tab4)
    return jnp.transpose(out_p, (2, 0, 1))
